# baseline (speedup 1.0000x reference)
import jax
import jax.numpy as jnp
from jax import lax
from jax.experimental import pallas as pl
from jax.experimental.pallas import tpu as pltpu


_DOT_TRANS_B = (((1,), (1,)), ((), ()))

_TN = 256
_NBUF = 4


def _noisy_body(x_hbm, mu_hbm, sig_hbm, eps_oc_ref, eps_in_ref,
                mu_b_ref, sig_b_ref, eps_or_ref, o_ref,
                x_vmem, x_bf, mu_ring, sig_ring,
                sem_x, sem_mu, sem_sig):
    F_out = mu_hbm.shape[0]
    nt = F_out // _TN
    j = pl.program_id(0)

    @pl.when(j == 0)
    def _prologue():
        pltpu.make_async_copy(x_hbm, x_vmem, sem_x).start()
        for t in range(min(_NBUF, nt)):
            sl = pl.ds(t * _TN, _TN)
            pltpu.make_async_copy(mu_hbm.at[sl, :], mu_ring.at[t],
                                  sem_mu.at[t]).start()
            pltpu.make_async_copy(sig_hbm.at[sl, :], sig_ring.at[t],
                                  sem_sig.at[t]).start()
        pltpu.make_async_copy(x_hbm, x_vmem, sem_x).wait()
        x_bf[...] = x_vmem[...].astype(jnp.bfloat16)

    slot = lax.rem(j, _NBUF)
    pltpu.make_async_copy(mu_ring.at[slot], mu_ring.at[slot],
                          sem_mu.at[slot]).wait()
    pltpu.make_async_copy(sig_ring.at[slot], sig_ring.at[slot],
                          sem_sig.at[slot]).wait()

    scale = eps_oc_ref[...] * eps_in_ref[...]
    w_eff = (mu_ring[slot] + sig_ring[slot] * scale).astype(jnp.bfloat16)
    y = lax.dot_general(x_bf[...], w_eff, _DOT_TRANS_B,
                        preferred_element_type=jnp.float32)
    b_eff = sig_b_ref[...] * eps_or_ref[...] + mu_b_ref[...]
    o_ref[...] = y + b_eff

    @pl.when(j + _NBUF < nt)
    def _refill():
        nsl = pl.ds((j + _NBUF) * _TN, _TN)
        pltpu.make_async_copy(mu_hbm.at[nsl, :], mu_ring.at[slot],
                              sem_mu.at[slot]).start()
        pltpu.make_async_copy(sig_hbm.at[nsl, :], sig_ring.at[slot],
                              sem_sig.at[slot]).start()


def kernel(x, mu_weight, sigma_weight, mu_bias, sigma_bias, eps_in, eps_out):
    B, F_in = x.shape
    F_out = mu_bias.shape[0]

    x_f = x.astype(jnp.float32)
    mu_w = mu_weight.astype(jnp.float32)
    sig_w = sigma_weight.astype(jnp.float32)
    eps_in_row = eps_in.reshape(1, F_in).astype(jnp.float32)
    eps_out_col = eps_out.reshape(F_out, 1).astype(jnp.float32)
    eps_out_row = eps_out.reshape(1, F_out).astype(jnp.float32)
    mu_b_row = mu_bias.reshape(1, F_out).astype(jnp.float32)
    sig_b_row = sigma_bias.reshape(1, F_out).astype(jnp.float32)

    any_spec = pl.BlockSpec(memory_space=pl.ANY)
    grid = (F_out // _TN,)

    return pl.pallas_call(
        _noisy_body,
        out_shape=jax.ShapeDtypeStruct((B, F_out), jnp.float32),
        grid=grid,
        in_specs=[
            any_spec,
            any_spec,
            any_spec,
            pl.BlockSpec((_TN, 1), lambda j: (j, 0)),
            pl.BlockSpec((1, F_in), lambda j: (0, 0)),
            pl.BlockSpec((1, _TN), lambda j: (0, j)),
            pl.BlockSpec((1, _TN), lambda j: (0, j)),
            pl.BlockSpec((1, _TN), lambda j: (0, j)),
        ],
        out_specs=pl.BlockSpec((B, _TN), lambda j: (0, j)),
        scratch_shapes=[
            pltpu.VMEM((B, F_in), jnp.float32),
            pltpu.VMEM((B, F_in), jnp.bfloat16),
            pltpu.VMEM((_NBUF, _TN, F_in), jnp.float32),
            pltpu.VMEM((_NBUF, _TN, F_in), jnp.float32),
            pltpu.SemaphoreType.DMA,
            pltpu.SemaphoreType.DMA((_NBUF,)),
            pltpu.SemaphoreType.DMA((_NBUF,)),
        ],
        compiler_params=pltpu.CompilerParams(
            dimension_semantics=("arbitrary",),
            vmem_limit_bytes=64 * 1024 * 1024,
        ),
    )(x_f, mu_w, sig_w, eps_out_col, eps_in_row, mu_b_row, sig_b_row,
      eps_out_row)
